# baseline (speedup 1.0000x reference)
import functools

import jax
import jax.numpy as jnp
from jax import lax
from jax.experimental import pallas as pl
from jax.experimental.pallas import tpu as pltpu
from jax.experimental.pallas import tpu_sc as plsc

N = 10000
E = 320000
D = 128
EPS = 1e-5

NC = 2
NS = 16
L = 16

NP = 10240
DEG_ROWS = NP // NS
CH = 96
NCHUNK = 105
E_PER_TILE = NCHUNK * CH
E_PAD = NC * NS * E_PER_TILE
NPAD = 10112
AGG_ROWS = NPAD // NS

_mesh = plsc.VectorSubcoreMesh(core_axis_name="c", subcore_axis_name="s",
                               num_cores=NC, num_subcores=NS)


DSUP = 8
NSUP = NCHUNK // DSUP
DREM = NCHUNK - NSUP * DSUP


@functools.partial(
    pl.kernel,
    out_type=jax.ShapeDtypeStruct((NC * NP,), jnp.float32),
    mesh=_mesh,
    scratch_types=[
        pltpu.VMEM((DSUP, CH), jnp.int32),
        pltpu.VMEM((CH,), jnp.float32),
        pltpu.VMEM((DEG_ROWS,), jnp.float32),
        pltpu.VMEM_SHARED((NP,), jnp.float32),
        pltpu.SemaphoreType.DMA,
    ],
)
def _deg_kernel(dst_hbm, out_hbm, idx_v, ones_v, zb_v, acc_sh, dsem):
    c = lax.axis_index("c")
    s = lax.axis_index("s")
    wid = c * NS + s
    ones16 = jnp.ones((L,), jnp.float32)
    zeros16 = jnp.zeros((L,), jnp.float32)
    for k in range(CH // L):
        ones_v[pl.ds(k * L, L)] = ones16

    def _zf(i, carry):
        zb_v[pl.ds(i * L, L)] = zeros16
        return carry

    lax.fori_loop(0, DEG_ROWS // L, _zf, 0)
    pltpu.sync_copy(zb_v, acc_sh.at[pl.ds(s * DEG_ROWS, DEG_ROWS)])
    plsc.subcore_barrier()

    def _scat(j):
        pltpu.async_copy(ones_v, acc_sh.at[idx_v.at[j]], dsem, add=True)

    def _wait_one():
        pltpu.make_async_copy(ones_v, acc_sh.at[idx_v.at[0]], dsem).wait()

    def _sup(t, carry):
        pltpu.sync_copy(dst_hbm.at[wid, pl.ds(t * DSUP, DSUP)], idx_v)
        for j in range(DSUP):
            _scat(j)
        for j in range(DSUP):
            _wait_one()
        return carry

    lax.fori_loop(0, NSUP, _sup, 0)
    pltpu.sync_copy(dst_hbm.at[wid, pl.ds(NSUP * DSUP, DREM)],
                    idx_v.at[pl.ds(0, DREM)])
    for j in range(DREM):
        _scat(j)
    for j in range(DREM):
        _wait_one()
    plsc.subcore_barrier()
    pltpu.sync_copy(acc_sh.at[pl.ds(s * DEG_ROWS, DEG_ROWS)],
                    out_hbm.at[pl.ds(c * NP + s * DEG_ROWS, DEG_ROWS)])


NBUF = 3
NGRP = -(-NCHUNK // NBUF)
ZCOPY = AGG_ROWS // CH
ZREM = AGG_ROWS - ZCOPY * CH


@functools.partial(
    pl.kernel,
    out_type=jax.ShapeDtypeStruct((NC, NPAD, D), jnp.float32),
    mesh=_mesh,
    scratch_types=(
        [pltpu.VMEM((NCHUNK * CH,), jnp.int32),
         pltpu.VMEM((NBUF * CH,), jnp.int32),
         pltpu.VMEM((NBUF * CH,), jnp.int32),
         pltpu.VMEM((NBUF, CH, D), jnp.float32),
         pltpu.VMEM_SHARED((NPAD, D), jnp.float32)]
        + [pltpu.SemaphoreType.DMA] * (2 * NBUF)
    ),
)
def _agg_kernel(hp_hbm, pk_hbm, out_hbm, pk_v, st_s, st_d, rows, acc_sh,
                *sems):
    gsem = sems[:NBUF]
    ssem = sems[NBUF:]
    c = lax.axis_index("c")
    s = lax.axis_index("s")
    wid = c * NS + s
    zeros16 = jnp.zeros((L,), jnp.float32)

    pltpu.sync_copy(pk_hbm.at[wid], pk_v)

    def _zf(r, carry):
        for k in range(D // L):
            rows[0, r, pl.ds(k * L, L)] = zeros16
        return carry

    lax.fori_loop(0, CH, _zf, 0)
    for k in range(ZCOPY):
        pltpu.sync_copy(rows.at[0],
                        acc_sh.at[pl.ds(s * AGG_ROWS + k * CH, CH)])
    pltpu.sync_copy(rows.at[0].at[pl.ds(0, ZREM)],
                    acc_sh.at[pl.ds(s * AGG_ROWS + ZCOPY * CH, ZREM)])
    plsc.subcore_barrier()

    def _unpack(i, b):
        for k in range(CH // L):
            v = pk_v[pl.ds(i * CH + k * L, L)]
            st_s[pl.ds(b * CH + k * L, L)] = v & jnp.int32(0xFFFF)
            st_d[pl.ds(b * CH + k * L, L)] = lax.shift_right_logical(v, 16)

    def _fire_gather(b):
        pltpu.async_copy(hp_hbm.at[st_s.at[pl.ds(b * CH, CH)]],
                         rows.at[b], gsem[b])

    for b in range(NBUF):
        _unpack(b, b)
        _fire_gather(b)

    def _group(g, carry):
        i0 = g * NBUF
        for b in range(NBUF):
            @pl.when(i0 + b < NCHUNK)
            def _():
                pltpu.make_async_copy(
                    hp_hbm.at[st_s.at[pl.ds(b * CH, CH)]],
                    rows.at[b], gsem[b]).wait()
                pltpu.async_copy(
                    rows.at[b], acc_sh.at[st_d.at[pl.ds(b * CH, CH)]],
                    ssem[b], add=True)

        for b in range(NBUF):
            @pl.when(i0 + b < NCHUNK)
            def _():
                pltpu.make_async_copy(
                    rows.at[b], acc_sh.at[st_d.at[pl.ds(b * CH, CH)]],
                    ssem[b]).wait()

            nxt = i0 + NBUF + b

            @pl.when(nxt < NCHUNK)
            def _():
                _unpack(nxt, b)
                _fire_gather(b)

        return carry

    lax.fori_loop(0, NGRP, _group, 0)
    plsc.subcore_barrier()
    pltpu.sync_copy(acc_sh.at[pl.ds(s * AGG_ROWS, AGG_ROWS)],
                    out_hbm.at[c, pl.ds(s * AGG_ROWS, AGG_ROWS)])


R = 1000
G = N // R

_DOT = dict(preferred_element_type=jnp.float32,
            precision=jax.lax.Precision.HIGHEST)


def _pack_body(ei_ref, out_ref):
    out_ref[...] = ei_ref[0, :] | (ei_ref[1, :] << 16)


_pack_call = pl.pallas_call(
    _pack_body,
    out_shape=jax.ShapeDtypeStruct((E_PAD,), jnp.int32),
)


def _pre_body(x_ref, w_ref, degp_ref, hp_ref, dis_ref):
    deg = degp_ref[:, 0:1] + degp_ref[:, 1:2] + 1.0
    dis = lax.rsqrt(deg)
    hp_ref[...] = jnp.dot(x_ref[...], w_ref[...], **_DOT) * dis
    dis_ref[...] = dis


def _mid_body(p_ref, hp_ref, dis_ref, b_ref, g_ref, be_ref, w_ref, out_ref):
    dis = dis_ref[...]
    t = (p_ref[0] + p_ref[1] + hp_ref[...]) * dis + b_ref[...]
    mu = jnp.mean(t, axis=-1, keepdims=True)
    var = jnp.mean((t - mu) ** 2, axis=-1, keepdims=True)
    u = (t - mu) / jnp.sqrt(var + EPS) * g_ref[...] + be_ref[...]
    u = jnp.maximum(u, 0.0)
    out_ref[...] = jnp.dot(u, w_ref[...], **_DOT) * dis


def _fin_body(p_ref, hp_ref, dis_ref, b_ref, out_ref):
    out_ref[...] = ((p_ref[0] + p_ref[1] + hp_ref[...]) * dis_ref[...]
                    + b_ref[...])


def _row_spec(width):
    return pl.BlockSpec((R, width), lambda i: (i, 0))


_PART_SPEC = pl.BlockSpec((NC, R, D), lambda i: (0, i, 0))
_VEC_SPEC = pl.BlockSpec((D,), lambda i: (0,))
_W_SPEC = pl.BlockSpec((D, D), lambda i: (0, 0))

_pre_call = pl.pallas_call(
    _pre_body,
    grid=(G,),
    in_specs=[_row_spec(D), _W_SPEC, _row_spec(2)],
    out_specs=[_row_spec(D), _row_spec(1)],
    out_shape=[jax.ShapeDtypeStruct((N, D), jnp.float32),
               jax.ShapeDtypeStruct((N, 1), jnp.float32)],
)

_mid_call = pl.pallas_call(
    _mid_body,
    grid=(G,),
    in_specs=[_PART_SPEC, _row_spec(D), _row_spec(1),
              _VEC_SPEC, _VEC_SPEC, _VEC_SPEC, _W_SPEC],
    out_specs=_row_spec(D),
    out_shape=jax.ShapeDtypeStruct((N, D), jnp.float32),
)

_fin_call = pl.pallas_call(
    _fin_body,
    grid=(G,),
    in_specs=[_PART_SPEC, _row_spec(D), _row_spec(1), _VEC_SPEC],
    out_specs=_row_spec(D),
    out_shape=jax.ShapeDtypeStruct((N, D), jnp.float32),
)


_NT = NC * NS
_PPT = E_PER_TILE - E // _NT


def kernel(x, edge_index, W1, b1, g1, be1, W2, b2):
    ei3 = edge_index.reshape(2, _NT, E // _NT)
    if _PPT:
        pad_dst = (jnp.arange(_PPT, dtype=jnp.int32) % (NPAD - N)) + N
        pad3 = jnp.stack([jnp.zeros((_NT, _PPT), jnp.int32),
                          jnp.broadcast_to(pad_dst, (_NT, _PPT))])
        ei3 = jnp.concatenate([ei3, pad3], axis=2)
    dst3 = ei3[1].reshape(_NT, NCHUNK, CH)
    pk3 = _pack_call(ei3.reshape(2, E_PAD)).reshape(_NT, NCHUNK * CH)
    degp = _deg_kernel(dst3).reshape(NC, NP)
    degp_t = degp.T[:N]
    hp1, dis = _pre_call(x, W1, degp_t)
    p1 = _agg_kernel(hp1, pk3)
    hp2 = _mid_call(p1, hp1, dis, b1, g1, be1, W2)
    p2 = _agg_kernel(hp2, pk3)
    return _fin_call(p2, hp2, dis, b2)


# speedup vs baseline: 1.1427x; 1.1427x over previous
import functools

import jax
import jax.numpy as jnp
from jax import lax
from jax.experimental import pallas as pl
from jax.experimental.pallas import tpu as pltpu
from jax.experimental.pallas import tpu_sc as plsc

N = 10000
E = 320000
D = 128
EPS = 1e-5

NC = 2
NS = 16
L = 16

NP = 10240
DEG_ROWS = NP // NS
CH = 64
NCHUNK = 157
E_PER_TILE = NCHUNK * CH
E_PAD = NC * NS * E_PER_TILE
NPAD = 10112
AGG_ROWS = NPAD // NS

_mesh = plsc.VectorSubcoreMesh(core_axis_name="c", subcore_axis_name="s",
                               num_cores=NC, num_subcores=NS)


DSUP = 8
NSUP = NCHUNK // DSUP
DREM = NCHUNK - NSUP * DSUP


@functools.partial(
    pl.kernel,
    out_type=jax.ShapeDtypeStruct((NC * NP,), jnp.float32),
    mesh=_mesh,
    scratch_types=[
        pltpu.VMEM((DSUP, CH), jnp.int32),
        pltpu.VMEM((CH,), jnp.float32),
        pltpu.VMEM((DEG_ROWS,), jnp.float32),
        pltpu.VMEM_SHARED((NP,), jnp.float32),
        pltpu.SemaphoreType.DMA,
    ],
)
def _deg_kernel(dst_hbm, out_hbm, idx_v, ones_v, zb_v, acc_sh, dsem):
    c = lax.axis_index("c")
    s = lax.axis_index("s")
    wid = c * NS + s
    ones16 = jnp.ones((L,), jnp.float32)
    zeros16 = jnp.zeros((L,), jnp.float32)
    for k in range(CH // L):
        ones_v[pl.ds(k * L, L)] = ones16

    def _zf(i, carry):
        zb_v[pl.ds(i * L, L)] = zeros16
        return carry

    lax.fori_loop(0, DEG_ROWS // L, _zf, 0)
    pltpu.sync_copy(zb_v, acc_sh.at[pl.ds(s * DEG_ROWS, DEG_ROWS)])
    plsc.subcore_barrier()

    def _scat(j):
        pltpu.async_copy(ones_v, acc_sh.at[idx_v.at[j]], dsem, add=True)

    def _wait_one():
        pltpu.make_async_copy(ones_v, acc_sh.at[idx_v.at[0]], dsem).wait()

    def _sup(t, carry):
        pltpu.sync_copy(dst_hbm.at[wid, pl.ds(t * DSUP, DSUP)], idx_v)
        for j in range(DSUP):
            _scat(j)
        for j in range(DSUP):
            _wait_one()
        return carry

    lax.fori_loop(0, NSUP, _sup, 0)
    pltpu.sync_copy(dst_hbm.at[wid, pl.ds(NSUP * DSUP, DREM)],
                    idx_v.at[pl.ds(0, DREM)])
    for j in range(DREM):
        _scat(j)
    for j in range(DREM):
        _wait_one()
    plsc.subcore_barrier()
    pltpu.sync_copy(acc_sh.at[pl.ds(s * DEG_ROWS, DEG_ROWS)],
                    out_hbm.at[pl.ds(c * NP + s * DEG_ROWS, DEG_ROWS)])


NBUF = 3
NGRP = -(-NCHUNK // NBUF)
ZCOPY = AGG_ROWS // CH
ZREM = AGG_ROWS - ZCOPY * CH


@functools.partial(
    pl.kernel,
    out_type=jax.ShapeDtypeStruct((NC, NPAD, D), jnp.float32),
    mesh=_mesh,
    scratch_types=(
        [pltpu.VMEM((NCHUNK * CH,), jnp.int32),
         pltpu.VMEM((NBUF * CH,), jnp.int32),
         pltpu.VMEM((NBUF * CH,), jnp.int32),
         pltpu.VMEM((NBUF, CH, D), jnp.float32),
         pltpu.VMEM_SHARED((NPAD, D), jnp.float32)]
        + [pltpu.SemaphoreType.DMA] * (2 * NBUF)
    ),
)
def _agg_kernel(hp_hbm, pk_hbm, out_hbm, pk_v, st_s, st_d, rows, acc_sh,
                *sems):
    gsem = sems[:NBUF]
    ssem = sems[NBUF:]
    c = lax.axis_index("c")
    s = lax.axis_index("s")
    wid = c * NS + s
    zeros16 = jnp.zeros((L,), jnp.float32)

    pltpu.sync_copy(pk_hbm.at[wid], pk_v)

    def _zf(r, carry):
        for k in range(D // L):
            rows[0, r, pl.ds(k * L, L)] = zeros16
        return carry

    lax.fori_loop(0, CH, _zf, 0)
    for k in range(ZCOPY):
        pltpu.sync_copy(rows.at[0],
                        acc_sh.at[pl.ds(s * AGG_ROWS + k * CH, CH)])
    pltpu.sync_copy(rows.at[0].at[pl.ds(0, ZREM)],
                    acc_sh.at[pl.ds(s * AGG_ROWS + ZCOPY * CH, ZREM)])
    plsc.subcore_barrier()

    def _unpack(i, b):
        for k in range(CH // L):
            v = pk_v[pl.ds(i * CH + k * L, L)]
            st_s[pl.ds(b * CH + k * L, L)] = v & jnp.int32(0xFFFF)
            st_d[pl.ds(b * CH + k * L, L)] = lax.shift_right_logical(v, 16)

    def _fire_gather(b):
        pltpu.async_copy(hp_hbm.at[st_s.at[pl.ds(b * CH, CH)]],
                         rows.at[b], gsem[b])

    for b in range(NBUF):
        _unpack(b, b)
        _fire_gather(b)

    def _group(g, carry):
        i0 = g * NBUF
        for b in range(NBUF):
            @pl.when(i0 + b < NCHUNK)
            def _():
                pltpu.make_async_copy(
                    hp_hbm.at[st_s.at[pl.ds(b * CH, CH)]],
                    rows.at[b], gsem[b]).wait()
                pltpu.async_copy(
                    rows.at[b], acc_sh.at[st_d.at[pl.ds(b * CH, CH)]],
                    ssem[b], add=True)

        for b in range(NBUF):
            @pl.when(i0 + b < NCHUNK)
            def _():
                pltpu.make_async_copy(
                    rows.at[b], acc_sh.at[st_d.at[pl.ds(b * CH, CH)]],
                    ssem[b]).wait()

            nxt = i0 + NBUF + b

            @pl.when(nxt < NCHUNK)
            def _():
                _unpack(nxt, b)
                _fire_gather(b)

        return carry

    lax.fori_loop(0, NGRP, _group, 0)
    plsc.subcore_barrier()
    pltpu.sync_copy(acc_sh.at[pl.ds(s * AGG_ROWS, AGG_ROWS)],
                    out_hbm.at[c, pl.ds(s * AGG_ROWS, AGG_ROWS)])


R = 1000
G = N // R

_DOT = dict(preferred_element_type=jnp.float32,
            precision=jax.lax.Precision.HIGHEST)


def _pack_body(ei_ref, out_ref):
    out_ref[...] = ei_ref[0, :] | (ei_ref[1, :] << 16)


_pack_call = pl.pallas_call(
    _pack_body,
    out_shape=jax.ShapeDtypeStruct((E_PAD,), jnp.int32),
)


def _pre_body(x_ref, w_ref, degp_ref, hp_ref, dis_ref):
    deg = degp_ref[:, 0:1] + degp_ref[:, 1:2] + 1.0
    dis = lax.rsqrt(deg)
    hp_ref[...] = jnp.dot(x_ref[...], w_ref[...], **_DOT) * dis
    dis_ref[...] = dis


def _mid_body(p_ref, hp_ref, dis_ref, b_ref, g_ref, be_ref, w_ref, out_ref):
    dis = dis_ref[...]
    t = (p_ref[0] + p_ref[1] + hp_ref[...]) * dis + b_ref[...]
    mu = jnp.mean(t, axis=-1, keepdims=True)
    var = jnp.mean((t - mu) ** 2, axis=-1, keepdims=True)
    u = (t - mu) / jnp.sqrt(var + EPS) * g_ref[...] + be_ref[...]
    u = jnp.maximum(u, 0.0)
    out_ref[...] = jnp.dot(u, w_ref[...], **_DOT) * dis


def _fin_body(p_ref, hp_ref, dis_ref, b_ref, out_ref):
    out_ref[...] = ((p_ref[0] + p_ref[1] + hp_ref[...]) * dis_ref[...]
                    + b_ref[...])


def _row_spec(width):
    return pl.BlockSpec((R, width), lambda i: (i, 0))


_PART_SPEC = pl.BlockSpec((NC, R, D), lambda i: (0, i, 0))
_VEC_SPEC = pl.BlockSpec((D,), lambda i: (0,))
_W_SPEC = pl.BlockSpec((D, D), lambda i: (0, 0))

_pre_call = pl.pallas_call(
    _pre_body,
    grid=(G,),
    in_specs=[_row_spec(D), _W_SPEC, _row_spec(2)],
    out_specs=[_row_spec(D), _row_spec(1)],
    out_shape=[jax.ShapeDtypeStruct((N, D), jnp.float32),
               jax.ShapeDtypeStruct((N, 1), jnp.float32)],
)

_mid_call = pl.pallas_call(
    _mid_body,
    grid=(G,),
    in_specs=[_PART_SPEC, _row_spec(D), _row_spec(1),
              _VEC_SPEC, _VEC_SPEC, _VEC_SPEC, _W_SPEC],
    out_specs=_row_spec(D),
    out_shape=jax.ShapeDtypeStruct((N, D), jnp.float32),
)

_fin_call = pl.pallas_call(
    _fin_body,
    grid=(G,),
    in_specs=[_PART_SPEC, _row_spec(D), _row_spec(1), _VEC_SPEC],
    out_specs=_row_spec(D),
    out_shape=jax.ShapeDtypeStruct((N, D), jnp.float32),
)


_NT = NC * NS
_PPT = E_PER_TILE - E // _NT


def kernel(x, edge_index, W1, b1, g1, be1, W2, b2):
    ei3 = edge_index.reshape(2, _NT, E // _NT)
    if _PPT:
        pad_dst = (jnp.arange(_PPT, dtype=jnp.int32) % (NPAD - N)) + N
        pad3 = jnp.stack([jnp.zeros((_NT, _PPT), jnp.int32),
                          jnp.broadcast_to(pad_dst, (_NT, _PPT))])
        ei3 = jnp.concatenate([ei3, pad3], axis=2)
    dst3 = ei3[1].reshape(_NT, NCHUNK, CH)
    pk3 = _pack_call(ei3.reshape(2, E_PAD)).reshape(_NT, NCHUNK * CH)
    degp = _deg_kernel(dst3).reshape(NC, NP)
    degp_t = degp.T[:N]
    hp1, dis = _pre_call(x, W1, degp_t)
    p1 = _agg_kernel(hp1, pk3)
    hp2 = _mid_call(p1, hp1, dis, b1, g1, be1, W2)
    p2 = _agg_kernel(hp2, pk3)
    return _fin_call(p2, hp2, dis, b2)


# speedup vs baseline: 1.4786x; 1.2939x over previous
import functools

import jax
import jax.numpy as jnp
from jax import lax
from jax.experimental import pallas as pl
from jax.experimental.pallas import tpu as pltpu
from jax.experimental.pallas import tpu_sc as plsc

N = 10000
E = 320000
D = 128
EPS = 1e-5

NC = 2
NS = 16
L = 16

NP = 10240
DEG_ROWS = NP // NS
CH = 64
NCHUNK = 158
E_PER_TILE = NCHUNK * CH
E_PAD = NC * NS * E_PER_TILE
NPAD = 10112
AGG_ROWS = NPAD // NS

_mesh = plsc.VectorSubcoreMesh(core_axis_name="c", subcore_axis_name="s",
                               num_cores=NC, num_subcores=NS)


DSUP = 8
NSUP = NCHUNK // DSUP
DREM = NCHUNK - NSUP * DSUP


@functools.partial(
    pl.kernel,
    out_type=jax.ShapeDtypeStruct((NC * NP,), jnp.float32),
    mesh=_mesh,
    scratch_types=(
        [pltpu.VMEM((DSUP, CH), jnp.int32),
         pltpu.VMEM((CH,), jnp.float32),
         pltpu.VMEM((DEG_ROWS,), jnp.float32),
         pltpu.VMEM_SHARED((NP,), jnp.float32)]
        + [pltpu.SemaphoreType.DMA] * DSUP
    ),
)
def _deg_kernel(dst_hbm, out_hbm, idx_v, ones_v, zb_v, acc_sh, *dsems):
    c = lax.axis_index("c")
    s = lax.axis_index("s")
    wid = c * NS + s
    ones16 = jnp.ones((L,), jnp.float32)
    zeros16 = jnp.zeros((L,), jnp.float32)
    for k in range(CH // L):
        ones_v[pl.ds(k * L, L)] = ones16

    def _zf(i, carry):
        zb_v[pl.ds(i * L, L)] = zeros16
        return carry

    lax.fori_loop(0, DEG_ROWS // L, _zf, 0)
    pltpu.sync_copy(zb_v, acc_sh.at[pl.ds(s * DEG_ROWS, DEG_ROWS)])
    plsc.subcore_barrier()

    def _scat(j):
        pltpu.async_copy(ones_v, acc_sh.at[idx_v.at[j]], dsems[j], add=True)

    def _wait(j):
        pltpu.make_async_copy(ones_v, acc_sh.at[idx_v.at[j]],
                              dsems[j]).wait()

    def _sup(t, carry):
        pltpu.sync_copy(dst_hbm.at[wid, pl.ds(t * DSUP, DSUP)], idx_v)
        for h in range(2):
            for j in range(h * 4, h * 4 + 4):
                _scat(j)
            for j in range(h * 4, h * 4 + 4):
                _wait(j)
        return carry

    lax.fori_loop(0, NSUP, _sup, 0)
    pltpu.sync_copy(dst_hbm.at[wid, pl.ds(NSUP * DSUP, DREM)],
                    idx_v.at[pl.ds(0, DREM)])
    for j in range(DREM):
        _scat(j)
    for j in range(DREM):
        _wait(j)
    plsc.subcore_barrier()
    pltpu.sync_copy(acc_sh.at[pl.ds(s * DEG_ROWS, DEG_ROWS)],
                    out_hbm.at[pl.ds(c * NP + s * DEG_ROWS, DEG_ROWS)])


NBUF = 3
NGRP = -(-NCHUNK // NBUF)
ZCOPY = AGG_ROWS // CH
ZREM = AGG_ROWS - ZCOPY * CH


@functools.partial(
    pl.kernel,
    out_type=jax.ShapeDtypeStruct((NC, NPAD, D), jnp.float32),
    mesh=_mesh,
    scratch_types=(
        [pltpu.VMEM((NCHUNK * CH,), jnp.int32),
         pltpu.VMEM((NBUF * CH,), jnp.int32),
         pltpu.VMEM((NBUF * CH,), jnp.int32),
         pltpu.VMEM((NBUF, CH, D), jnp.float32),
         pltpu.VMEM_SHARED((NPAD, D), jnp.float32)]
        + [pltpu.SemaphoreType.DMA] * (2 * NBUF)
    ),
)
def _agg_kernel(hp_hbm, pk_hbm, out_hbm, pk_v, st_s, st_d, rows, acc_sh,
                *sems):
    gsem = sems[:NBUF]
    ssem = sems[NBUF:]
    c = lax.axis_index("c")
    s = lax.axis_index("s")
    wid = c * NS + s
    zeros16 = jnp.zeros((L,), jnp.float32)

    pltpu.sync_copy(pk_hbm.at[wid], pk_v)

    def _zf(r, carry):
        for k in range(D // L):
            rows[0, r, pl.ds(k * L, L)] = zeros16
        return carry

    lax.fori_loop(0, CH, _zf, 0)
    for k in range(ZCOPY):
        pltpu.sync_copy(rows.at[0],
                        acc_sh.at[pl.ds(s * AGG_ROWS + k * CH, CH)])
    pltpu.sync_copy(rows.at[0].at[pl.ds(0, ZREM)],
                    acc_sh.at[pl.ds(s * AGG_ROWS + ZCOPY * CH, ZREM)])
    plsc.subcore_barrier()

    def _unpack(i, b):
        for k in range(CH // L):
            v = pk_v[pl.ds(i * CH + k * L, L)]
            st_s[pl.ds(b * CH + k * L, L)] = v & jnp.int32(0xFFFF)
            st_d[pl.ds(b * CH + k * L, L)] = lax.shift_right_logical(v, 16)

    def _fire_gather(b):
        pltpu.async_copy(hp_hbm.at[st_s.at[pl.ds(b * CH, CH)]],
                         rows.at[b], gsem[b])

    for b in range(NBUF):
        _unpack(b, b)
        _fire_gather(b)

    def _group(g, carry):
        i0 = g * NBUF
        for b in range(NBUF):
            @pl.when(i0 + b < NCHUNK)
            def _():
                pltpu.make_async_copy(
                    hp_hbm.at[st_s.at[pl.ds(b * CH, CH)]],
                    rows.at[b], gsem[b]).wait()
                pltpu.async_copy(
                    rows.at[b], acc_sh.at[st_d.at[pl.ds(b * CH, CH)]],
                    ssem[b], add=True)

        for b in range(NBUF):
            @pl.when(i0 + b < NCHUNK)
            def _():
                pltpu.make_async_copy(
                    rows.at[b], acc_sh.at[st_d.at[pl.ds(b * CH, CH)]],
                    ssem[b]).wait()

            nxt = i0 + NBUF + b

            @pl.when(nxt < NCHUNK)
            def _():
                _unpack(nxt, b)
                _fire_gather(b)

        return carry

    lax.fori_loop(0, NGRP, _group, 0)
    plsc.subcore_barrier()
    pltpu.sync_copy(acc_sh.at[pl.ds(s * AGG_ROWS, AGG_ROWS)],
                    out_hbm.at[c, pl.ds(s * AGG_ROWS, AGG_ROWS)])


R = 1000
G = N // R

_DOT = dict(preferred_element_type=jnp.float32,
            precision=jax.lax.Precision.HIGHEST)


def _pack_body(ei_ref, out_ref):
    out_ref[...] = ei_ref[0, :] | (ei_ref[1, :] << 16)


_pack_call = pl.pallas_call(
    _pack_body,
    out_shape=jax.ShapeDtypeStruct((E_PAD,), jnp.int32),
)


def _pre_body(x_ref, w_ref, degp_ref, hp_ref, dis_ref):
    deg = degp_ref[:, 0:1] + degp_ref[:, 1:2] + 1.0
    dis = lax.rsqrt(deg)
    hp_ref[...] = jnp.dot(x_ref[...], w_ref[...], **_DOT) * dis
    dis_ref[...] = dis


def _mid_body(p_ref, hp_ref, dis_ref, b_ref, g_ref, be_ref, w_ref, out_ref):
    dis = dis_ref[...]
    t = (p_ref[0] + p_ref[1] + hp_ref[...]) * dis + b_ref[...]
    mu = jnp.mean(t, axis=-1, keepdims=True)
    var = jnp.mean((t - mu) ** 2, axis=-1, keepdims=True)
    u = (t - mu) / jnp.sqrt(var + EPS) * g_ref[...] + be_ref[...]
    u = jnp.maximum(u, 0.0)
    out_ref[...] = jnp.dot(u, w_ref[...], **_DOT) * dis


def _fin_body(p_ref, hp_ref, dis_ref, b_ref, out_ref):
    out_ref[...] = ((p_ref[0] + p_ref[1] + hp_ref[...]) * dis_ref[...]
                    + b_ref[...])


def _row_spec(width):
    return pl.BlockSpec((R, width), lambda i: (i, 0))


_PART_SPEC = pl.BlockSpec((NC, R, D), lambda i: (0, i, 0))
_VEC_SPEC = pl.BlockSpec((D,), lambda i: (0,))
_W_SPEC = pl.BlockSpec((D, D), lambda i: (0, 0))

_pre_call = pl.pallas_call(
    _pre_body,
    grid=(G,),
    in_specs=[_row_spec(D), _W_SPEC, _row_spec(2)],
    out_specs=[_row_spec(D), _row_spec(1)],
    out_shape=[jax.ShapeDtypeStruct((N, D), jnp.float32),
               jax.ShapeDtypeStruct((N, 1), jnp.float32)],
)

_mid_call = pl.pallas_call(
    _mid_body,
    grid=(G,),
    in_specs=[_PART_SPEC, _row_spec(D), _row_spec(1),
              _VEC_SPEC, _VEC_SPEC, _VEC_SPEC, _W_SPEC],
    out_specs=_row_spec(D),
    out_shape=jax.ShapeDtypeStruct((N, D), jnp.float32),
)

_fin_call = pl.pallas_call(
    _fin_body,
    grid=(G,),
    in_specs=[_PART_SPEC, _row_spec(D), _row_spec(1), _VEC_SPEC],
    out_specs=_row_spec(D),
    out_shape=jax.ShapeDtypeStruct((N, D), jnp.float32),
)


_NT = NC * NS
_PPT = E_PER_TILE - E // _NT


def kernel(x, edge_index, W1, b1, g1, be1, W2, b2):
    ei3 = edge_index.reshape(2, _NT, E // _NT)
    if _PPT:
        pad_dst = (jnp.arange(_PPT, dtype=jnp.int32) % (NPAD - N)) + N
        pad_src = (jnp.arange(_PPT, dtype=jnp.int32) * 131) % N
        pad3 = jnp.stack([jnp.broadcast_to(pad_src, (_NT, _PPT)),
                          jnp.broadcast_to(pad_dst, (_NT, _PPT))])
        ei3 = jnp.concatenate([ei3, pad3], axis=2)
    dst3 = ei3[1].reshape(_NT, NCHUNK, CH)
    pk3 = _pack_call(ei3.reshape(2, E_PAD)).reshape(_NT, NCHUNK * CH)
    degp = _deg_kernel(dst3).reshape(NC, NP)
    degp_t = degp.T[:N]
    hp1, dis = _pre_call(x, W1, degp_t)
    p1 = _agg_kernel(hp1, pk3)
    hp2 = _mid_call(p1, hp1, dis, b1, g1, be1, W2)
    p2 = _agg_kernel(hp2, pk3)
    return _fin_call(p2, hp2, dis, b2)


# speedup vs baseline: 1.4847x; 1.0042x over previous
import functools

import jax
import jax.numpy as jnp
from jax import lax
from jax.experimental import pallas as pl
from jax.experimental.pallas import tpu as pltpu
from jax.experimental.pallas import tpu_sc as plsc

N = 10000
E = 320000
D = 128
EPS = 1e-5

NC = 2
NS = 16
L = 16

NP = 10240
DEG_ROWS = NP // NS
CH = 64
NCHUNK = 158
E_PER_TILE = NCHUNK * CH
E_PAD = NC * NS * E_PER_TILE
NPAD = 10112
AGG_ROWS = NPAD // NS

_mesh = plsc.VectorSubcoreMesh(core_axis_name="c", subcore_axis_name="s",
                               num_cores=NC, num_subcores=NS)


DSUP = 8
NSUP = NCHUNK // DSUP
DREM = NCHUNK - NSUP * DSUP


@functools.partial(
    pl.kernel,
    out_type=jax.ShapeDtypeStruct((NC * NP,), jnp.float32),
    mesh=_mesh,
    scratch_types=(
        [pltpu.VMEM((DSUP, CH), jnp.int32),
         pltpu.VMEM((CH,), jnp.float32),
         pltpu.VMEM((DEG_ROWS,), jnp.float32),
         pltpu.VMEM_SHARED((NP,), jnp.float32)]
        + [pltpu.SemaphoreType.DMA] * DSUP
    ),
)
def _deg_kernel(dst_hbm, out_hbm, idx_v, ones_v, zb_v, acc_sh, *dsems):
    c = lax.axis_index("c")
    s = lax.axis_index("s")
    wid = c * NS + s
    ones16 = jnp.ones((L,), jnp.float32)
    zeros16 = jnp.zeros((L,), jnp.float32)
    for k in range(CH // L):
        ones_v[pl.ds(k * L, L)] = ones16

    def _zf(i, carry):
        zb_v[pl.ds(i * L, L)] = zeros16
        return carry

    lax.fori_loop(0, DEG_ROWS // L, _zf, 0)
    pltpu.sync_copy(zb_v, acc_sh.at[pl.ds(s * DEG_ROWS, DEG_ROWS)])
    plsc.subcore_barrier()

    def _scat(j):
        pltpu.async_copy(ones_v, acc_sh.at[idx_v.at[j]], dsems[j], add=True)

    def _wait(j):
        pltpu.make_async_copy(ones_v, acc_sh.at[idx_v.at[j]],
                              dsems[j]).wait()

    def _sup(t, carry):
        pltpu.sync_copy(dst_hbm.at[wid, pl.ds(t * DSUP, DSUP)], idx_v)
        for h in range(2):
            for j in range(h * 4, h * 4 + 4):
                _scat(j)
            for j in range(h * 4, h * 4 + 4):
                _wait(j)
        return carry

    lax.fori_loop(0, NSUP, _sup, 0)
    pltpu.sync_copy(dst_hbm.at[wid, pl.ds(NSUP * DSUP, DREM)],
                    idx_v.at[pl.ds(0, DREM)])
    for j in range(DREM):
        _scat(j)
    for j in range(DREM):
        _wait(j)
    plsc.subcore_barrier()
    pltpu.sync_copy(acc_sh.at[pl.ds(s * DEG_ROWS, DEG_ROWS)],
                    out_hbm.at[pl.ds(c * NP + s * DEG_ROWS, DEG_ROWS)])


NBUF = 3
NGRP = -(-NCHUNK // NBUF)
ZCOPY = AGG_ROWS // CH
ZREM = AGG_ROWS - ZCOPY * CH


@functools.partial(
    pl.kernel,
    out_type=jax.ShapeDtypeStruct((NC, NPAD, D), jnp.float32),
    mesh=_mesh,
    scratch_types=(
        [pltpu.VMEM((NCHUNK * CH,), jnp.int32),
         pltpu.VMEM((NBUF * CH,), jnp.int32),
         pltpu.VMEM((NBUF * CH,), jnp.int32),
         pltpu.VMEM((NBUF, CH, D), jnp.float32),
         pltpu.VMEM_SHARED((NPAD, D), jnp.float32)]
        + [pltpu.SemaphoreType.DMA] * (2 * NBUF)
    ),
)
def _agg_kernel(hp_hbm, pk_hbm, out_hbm, pk_v, st_s, st_d, rows, acc_sh,
                *sems):
    gsem = sems[:NBUF]
    ssem = sems[NBUF:]
    c = lax.axis_index("c")
    s = lax.axis_index("s")
    wid = c * NS + s
    zeros16 = jnp.zeros((L,), jnp.float32)

    pltpu.sync_copy(pk_hbm.at[wid], pk_v)

    def _zf(r, carry):
        for k in range(D // L):
            rows[0, r, pl.ds(k * L, L)] = zeros16
        return carry

    lax.fori_loop(0, CH, _zf, 0)
    for k in range(ZCOPY):
        pltpu.sync_copy(rows.at[0],
                        acc_sh.at[pl.ds(s * AGG_ROWS + k * CH, CH)])
    pltpu.sync_copy(rows.at[0].at[pl.ds(0, ZREM)],
                    acc_sh.at[pl.ds(s * AGG_ROWS + ZCOPY * CH, ZREM)])
    plsc.subcore_barrier()

    def _unpack(i, b):
        for k in range(CH // L):
            v = pk_v[pl.ds(i * CH + k * L, L)]
            st_s[pl.ds(b * CH + k * L, L)] = v & jnp.int32(0xFFFF)
            st_d[pl.ds(b * CH + k * L, L)] = lax.shift_right_logical(v, 16)

    def _fire_gather(b):
        pltpu.async_copy(hp_hbm.at[st_s.at[pl.ds(b * CH, CH)]],
                         rows.at[b], gsem[b])

    for b in range(NBUF):
        _unpack(b, b)
        _fire_gather(b)

    def _group(g, carry):
        i0 = g * NBUF
        for b in range(NBUF):
            @pl.when(i0 + b < NCHUNK)
            def _():
                pltpu.make_async_copy(
                    hp_hbm.at[st_s.at[pl.ds(b * CH, CH)]],
                    rows.at[b], gsem[b]).wait()
                pltpu.async_copy(
                    rows.at[b], acc_sh.at[st_d.at[pl.ds(b * CH, CH)]],
                    ssem[b], add=True)

        for b in range(NBUF):
            @pl.when(i0 + b < NCHUNK)
            def _():
                pltpu.make_async_copy(
                    rows.at[b], acc_sh.at[st_d.at[pl.ds(b * CH, CH)]],
                    ssem[b]).wait()

            nxt = i0 + NBUF + b

            @pl.when(nxt < NCHUNK)
            def _():
                _unpack(nxt, b)
                _fire_gather(b)

        return carry

    lax.fori_loop(0, NGRP, _group, 0)
    plsc.subcore_barrier()
    pltpu.sync_copy(acc_sh.at[pl.ds(s * AGG_ROWS, AGG_ROWS)],
                    out_hbm.at[c, pl.ds(s * AGG_ROWS, AGG_ROWS)])


R = 1000
G = N // R

_DOT = dict(preferred_element_type=jnp.float32,
            precision=jax.lax.Precision.HIGHEST)


def _pack_body(ei_ref, out_ref):
    out_ref[...] = ei_ref[0, :] | (ei_ref[1, :] << 16)


_pack_call = pl.pallas_call(
    _pack_body,
    out_shape=jax.ShapeDtypeStruct((E_PAD,), jnp.int32),
)


def _mm_body(x_ref, w_ref, h_ref):
    h_ref[...] = jnp.dot(x_ref[...], w_ref[...], **_DOT)


def _scale_body(h_ref, degp_ref, hp_ref, dis_ref):
    deg = degp_ref[:, 0:1] + degp_ref[:, 1:2] + 1.0
    dis = lax.rsqrt(deg)
    hp_ref[...] = h_ref[...] * dis
    dis_ref[...] = dis


def _mid_body(p_ref, hp_ref, dis_ref, b_ref, g_ref, be_ref, w_ref, out_ref):
    dis = dis_ref[...]
    t = (p_ref[0] + p_ref[1] + hp_ref[...]) * dis + b_ref[...]
    mu = jnp.mean(t, axis=-1, keepdims=True)
    var = jnp.mean((t - mu) ** 2, axis=-1, keepdims=True)
    u = (t - mu) / jnp.sqrt(var + EPS) * g_ref[...] + be_ref[...]
    u = jnp.maximum(u, 0.0)
    out_ref[...] = jnp.dot(u, w_ref[...], **_DOT) * dis


def _fin_body(p_ref, hp_ref, dis_ref, b_ref, out_ref):
    out_ref[...] = ((p_ref[0] + p_ref[1] + hp_ref[...]) * dis_ref[...]
                    + b_ref[...])


def _row_spec(width):
    return pl.BlockSpec((R, width), lambda i: (i, 0))


_PART_SPEC = pl.BlockSpec((NC, R, D), lambda i: (0, i, 0))
_VEC_SPEC = pl.BlockSpec((D,), lambda i: (0,))
_W_SPEC = pl.BlockSpec((D, D), lambda i: (0, 0))

_mm_call = pl.pallas_call(
    _mm_body,
    grid=(G,),
    in_specs=[_row_spec(D), _W_SPEC],
    out_specs=_row_spec(D),
    out_shape=jax.ShapeDtypeStruct((N, D), jnp.float32),
)

_scale_call = pl.pallas_call(
    _scale_body,
    grid=(G,),
    in_specs=[_row_spec(D), _row_spec(2)],
    out_specs=[_row_spec(D), _row_spec(1)],
    out_shape=[jax.ShapeDtypeStruct((N, D), jnp.float32),
               jax.ShapeDtypeStruct((N, 1), jnp.float32)],
)

_mid_call = pl.pallas_call(
    _mid_body,
    grid=(G,),
    in_specs=[_PART_SPEC, _row_spec(D), _row_spec(1),
              _VEC_SPEC, _VEC_SPEC, _VEC_SPEC, _W_SPEC],
    out_specs=_row_spec(D),
    out_shape=jax.ShapeDtypeStruct((N, D), jnp.float32),
)

_fin_call = pl.pallas_call(
    _fin_body,
    grid=(G,),
    in_specs=[_PART_SPEC, _row_spec(D), _row_spec(1), _VEC_SPEC],
    out_specs=_row_spec(D),
    out_shape=jax.ShapeDtypeStruct((N, D), jnp.float32),
)


_NT = NC * NS
_PPT = E_PER_TILE - E // _NT


def kernel(x, edge_index, W1, b1, g1, be1, W2, b2):
    ei3 = edge_index.reshape(2, _NT, E // _NT)
    if _PPT:
        pad_dst = (jnp.arange(_PPT, dtype=jnp.int32) % (NPAD - N)) + N
        pad_src = (jnp.arange(_PPT, dtype=jnp.int32) * 131) % N
        pad3 = jnp.stack([jnp.broadcast_to(pad_src, (_NT, _PPT)),
                          jnp.broadcast_to(pad_dst, (_NT, _PPT))])
        ei3 = jnp.concatenate([ei3, pad3], axis=2)
    dst3 = ei3[1].reshape(_NT, NCHUNK, CH)
    pk3 = _pack_call(ei3.reshape(2, E_PAD)).reshape(_NT, NCHUNK * CH)
    degp = _deg_kernel(dst3).reshape(NC, NP)
    h1 = _mm_call(x, W1)
    degp_t = degp.T[:N]
    hp1, dis = _scale_call(h1, degp_t)
    p1 = _agg_kernel(hp1, pk3)
    hp2 = _mid_call(p1, hp1, dis, b1, g1, be1, W2)
    p2 = _agg_kernel(hp2, pk3)
    return _fin_call(p2, hp2, dis, b2)


# speedup vs baseline: 1.4897x; 1.0033x over previous
import functools

import jax
import jax.numpy as jnp
from jax import lax
from jax.experimental import pallas as pl
from jax.experimental.pallas import tpu as pltpu
from jax.experimental.pallas import tpu_sc as plsc

N = 10000
E = 320000
D = 128
EPS = 1e-5

NC = 2
NS = 16
L = 16

NP = 10240
DEG_ROWS = NP // NS
CH = 80
NCHUNK = 127
E_PER_TILE = NCHUNK * CH
E_PAD = NC * NS * E_PER_TILE
NPAD = 10112
AGG_ROWS = NPAD // NS

_mesh = plsc.VectorSubcoreMesh(core_axis_name="c", subcore_axis_name="s",
                               num_cores=NC, num_subcores=NS)


DSUP = 8
NSUP = NCHUNK // DSUP
DREM = NCHUNK - NSUP * DSUP


@functools.partial(
    pl.kernel,
    out_type=jax.ShapeDtypeStruct((NC * NP,), jnp.float32),
    mesh=_mesh,
    scratch_types=(
        [pltpu.VMEM((DSUP, CH), jnp.int32),
         pltpu.VMEM((CH,), jnp.float32),
         pltpu.VMEM((DEG_ROWS,), jnp.float32),
         pltpu.VMEM_SHARED((NP,), jnp.float32)]
        + [pltpu.SemaphoreType.DMA] * DSUP
    ),
)
def _deg_kernel(dst_hbm, out_hbm, idx_v, ones_v, zb_v, acc_sh, *dsems):
    c = lax.axis_index("c")
    s = lax.axis_index("s")
    wid = c * NS + s
    ones16 = jnp.ones((L,), jnp.float32)
    zeros16 = jnp.zeros((L,), jnp.float32)
    for k in range(CH // L):
        ones_v[pl.ds(k * L, L)] = ones16

    def _zf(i, carry):
        zb_v[pl.ds(i * L, L)] = zeros16
        return carry

    lax.fori_loop(0, DEG_ROWS // L, _zf, 0)
    pltpu.sync_copy(zb_v, acc_sh.at[pl.ds(s * DEG_ROWS, DEG_ROWS)])
    plsc.subcore_barrier()

    def _scat(j):
        pltpu.async_copy(ones_v, acc_sh.at[idx_v.at[j]], dsems[j], add=True)

    def _wait(j):
        pltpu.make_async_copy(ones_v, acc_sh.at[idx_v.at[j]],
                              dsems[j]).wait()

    def _sup(t, carry):
        pltpu.sync_copy(dst_hbm.at[wid, pl.ds(t * DSUP, DSUP)], idx_v)
        for h in range(2):
            for j in range(h * 4, h * 4 + 4):
                _scat(j)
            for j in range(h * 4, h * 4 + 4):
                _wait(j)
        return carry

    lax.fori_loop(0, NSUP, _sup, 0)
    pltpu.sync_copy(dst_hbm.at[wid, pl.ds(NSUP * DSUP, DREM)],
                    idx_v.at[pl.ds(0, DREM)])
    for j in range(DREM):
        _scat(j)
    for j in range(DREM):
        _wait(j)
    plsc.subcore_barrier()
    pltpu.sync_copy(acc_sh.at[pl.ds(s * DEG_ROWS, DEG_ROWS)],
                    out_hbm.at[pl.ds(c * NP + s * DEG_ROWS, DEG_ROWS)])


NBUF = 3
NGRP = -(-NCHUNK // NBUF)
ZCOPY = AGG_ROWS // CH
ZREM = AGG_ROWS - ZCOPY * CH


@functools.partial(
    pl.kernel,
    out_type=jax.ShapeDtypeStruct((NC, NPAD, D), jnp.float32),
    mesh=_mesh,
    scratch_types=(
        [pltpu.VMEM((NCHUNK * CH,), jnp.int32),
         pltpu.VMEM((NBUF * CH,), jnp.int32),
         pltpu.VMEM((NBUF * CH,), jnp.int32),
         pltpu.VMEM((NBUF, CH, D), jnp.float32),
         pltpu.VMEM_SHARED((NPAD, D), jnp.float32)]
        + [pltpu.SemaphoreType.DMA] * (2 * NBUF)
    ),
)
def _agg_kernel(hp_hbm, pk_hbm, out_hbm, pk_v, st_s, st_d, rows, acc_sh,
                *sems):
    gsem = sems[:NBUF]
    ssem = sems[NBUF:]
    c = lax.axis_index("c")
    s = lax.axis_index("s")
    wid = c * NS + s
    zeros16 = jnp.zeros((L,), jnp.float32)

    pltpu.sync_copy(pk_hbm.at[wid], pk_v)

    def _zf(r, carry):
        for k in range(D // L):
            rows[0, r, pl.ds(k * L, L)] = zeros16
        return carry

    lax.fori_loop(0, CH, _zf, 0)
    for k in range(ZCOPY):
        pltpu.sync_copy(rows.at[0],
                        acc_sh.at[pl.ds(s * AGG_ROWS + k * CH, CH)])
    pltpu.sync_copy(rows.at[0].at[pl.ds(0, ZREM)],
                    acc_sh.at[pl.ds(s * AGG_ROWS + ZCOPY * CH, ZREM)])
    plsc.subcore_barrier()

    def _unpack(i, b):
        for k in range(CH // L):
            v = pk_v[pl.ds(i * CH + k * L, L)]
            st_s[pl.ds(b * CH + k * L, L)] = v & jnp.int32(0xFFFF)
            st_d[pl.ds(b * CH + k * L, L)] = lax.shift_right_logical(v, 16)

    def _fire_gather(b):
        pltpu.async_copy(hp_hbm.at[st_s.at[pl.ds(b * CH, CH)]],
                         rows.at[b], gsem[b])

    for b in range(NBUF):
        _unpack(b, b)
        _fire_gather(b)

    def _group(g, carry):
        i0 = g * NBUF
        for b in range(NBUF):
            @pl.when(i0 + b < NCHUNK)
            def _():
                pltpu.make_async_copy(
                    hp_hbm.at[st_s.at[pl.ds(b * CH, CH)]],
                    rows.at[b], gsem[b]).wait()
                pltpu.async_copy(
                    rows.at[b], acc_sh.at[st_d.at[pl.ds(b * CH, CH)]],
                    ssem[b], add=True)

        for b in range(NBUF):
            @pl.when(i0 + b < NCHUNK)
            def _():
                pltpu.make_async_copy(
                    rows.at[b], acc_sh.at[st_d.at[pl.ds(b * CH, CH)]],
                    ssem[b]).wait()

            nxt = i0 + NBUF + b

            @pl.when(nxt < NCHUNK)
            def _():
                _unpack(nxt, b)
                _fire_gather(b)

        return carry

    lax.fori_loop(0, NGRP, _group, 0)
    plsc.subcore_barrier()
    pltpu.sync_copy(acc_sh.at[pl.ds(s * AGG_ROWS, AGG_ROWS)],
                    out_hbm.at[c, pl.ds(s * AGG_ROWS, AGG_ROWS)])


R = 1000
G = N // R

_DOT = dict(preferred_element_type=jnp.float32,
            precision=jax.lax.Precision.HIGHEST)


def _pack_body(ei_ref, out_ref):
    out_ref[...] = ei_ref[0, :] | (ei_ref[1, :] << 16)


_pack_call = pl.pallas_call(
    _pack_body,
    out_shape=jax.ShapeDtypeStruct((E_PAD,), jnp.int32),
)


def _mm_body(x_ref, w_ref, h_ref):
    h_ref[...] = jnp.dot(x_ref[...], w_ref[...], **_DOT)


def _scale_body(h_ref, degp_ref, hp_ref, dis_ref):
    deg = degp_ref[:, 0:1] + degp_ref[:, 1:2] + 1.0
    dis = lax.rsqrt(deg)
    hp_ref[...] = h_ref[...] * dis
    dis_ref[...] = dis


def _mid_body(p_ref, hp_ref, dis_ref, b_ref, g_ref, be_ref, w_ref, out_ref):
    dis = dis_ref[...]
    t = (p_ref[0] + p_ref[1] + hp_ref[...]) * dis + b_ref[...]
    mu = jnp.mean(t, axis=-1, keepdims=True)
    var = jnp.mean((t - mu) ** 2, axis=-1, keepdims=True)
    u = (t - mu) / jnp.sqrt(var + EPS) * g_ref[...] + be_ref[...]
    u = jnp.maximum(u, 0.0)
    out_ref[...] = jnp.dot(u, w_ref[...], **_DOT) * dis


def _fin_body(p_ref, hp_ref, dis_ref, b_ref, out_ref):
    out_ref[...] = ((p_ref[0] + p_ref[1] + hp_ref[...]) * dis_ref[...]
                    + b_ref[...])


def _row_spec(width):
    return pl.BlockSpec((R, width), lambda i: (i, 0))


_PART_SPEC = pl.BlockSpec((NC, R, D), lambda i: (0, i, 0))
_VEC_SPEC = pl.BlockSpec((D,), lambda i: (0,))
_W_SPEC = pl.BlockSpec((D, D), lambda i: (0, 0))

_mm_call = pl.pallas_call(
    _mm_body,
    grid=(G,),
    in_specs=[_row_spec(D), _W_SPEC],
    out_specs=_row_spec(D),
    out_shape=jax.ShapeDtypeStruct((N, D), jnp.float32),
)

_scale_call = pl.pallas_call(
    _scale_body,
    grid=(G,),
    in_specs=[_row_spec(D), _row_spec(2)],
    out_specs=[_row_spec(D), _row_spec(1)],
    out_shape=[jax.ShapeDtypeStruct((N, D), jnp.float32),
               jax.ShapeDtypeStruct((N, 1), jnp.float32)],
)

_mid_call = pl.pallas_call(
    _mid_body,
    grid=(G,),
    in_specs=[_PART_SPEC, _row_spec(D), _row_spec(1),
              _VEC_SPEC, _VEC_SPEC, _VEC_SPEC, _W_SPEC],
    out_specs=_row_spec(D),
    out_shape=jax.ShapeDtypeStruct((N, D), jnp.float32),
)

_fin_call = pl.pallas_call(
    _fin_body,
    grid=(G,),
    in_specs=[_PART_SPEC, _row_spec(D), _row_spec(1), _VEC_SPEC],
    out_specs=_row_spec(D),
    out_shape=jax.ShapeDtypeStruct((N, D), jnp.float32),
)


_NT = NC * NS
_PPT = E_PER_TILE - E // _NT


def kernel(x, edge_index, W1, b1, g1, be1, W2, b2):
    ei3 = edge_index.reshape(2, _NT, E // _NT)
    if _PPT:
        pad_dst = (jnp.arange(_PPT, dtype=jnp.int32) % (NPAD - N)) + N
        pad_src = (jnp.arange(_PPT, dtype=jnp.int32) * 131) % N
        pad3 = jnp.stack([jnp.broadcast_to(pad_src, (_NT, _PPT)),
                          jnp.broadcast_to(pad_dst, (_NT, _PPT))])
        ei3 = jnp.concatenate([ei3, pad3], axis=2)
    dst3 = ei3[1].reshape(_NT, NCHUNK, CH)
    pk3 = _pack_call(ei3.reshape(2, E_PAD)).reshape(_NT, NCHUNK * CH)
    degp = _deg_kernel(dst3).reshape(NC, NP)
    h1 = _mm_call(x, W1)
    degp_t = degp.T[:N]
    hp1, dis = _scale_call(h1, degp_t)
    p1 = _agg_kernel(hp1, pk3)
    hp2 = _mid_call(p1, hp1, dis, b1, g1, be1, W2)
    p2 = _agg_kernel(hp2, pk3)
    return _fin_call(p2, hp2, dis, b2)
